# baseline (speedup 1.0000x reference)
import functools

import jax
import jax.numpy as jnp
from jax import lax
from jax.experimental import pallas as pl
from jax.experimental.pallas import tpu as pltpu
from jax.experimental.pallas import tpu_sc as plsc

VOCAB = 1000000
DIM = 64
B = 4096
L = 200
N = B * L

WBLK = 16384
HBLK = WBLK // 2
NBLK = (VOCAB + WBLK - 1) // WBLK
VPAD = NBLK * WBLK

_info = plsc.get_sparse_core_info()
NC, NS = _info.num_cores, _info.num_subcores
NW = NC * NS
B_PER_W = B // NW
KTILES = L * DIM // 128
GP = KTILES + 1
SRC_ROWS = B * 2 * GP
NBUF = 4
ROUNDS = B_PER_W // NBUF


def _transpose_body(tt_ref, out_ref):
    a = tt_ref[:, :HBLK]
    b = tt_ref[:, HBLK:]
    c = jnp.concatenate([a, b], axis=0)
    out_ref[...] = c.T


_pack_table = pl.pallas_call(
    _transpose_body,
    grid=(NBLK,),
    in_specs=[pl.BlockSpec((DIM, WBLK), lambda q: (0, q))],
    out_specs=pl.BlockSpec((HBLK, 2 * DIM), lambda q: (q, 0)),
    out_shape=jax.ShapeDtypeStruct((VPAD // 2, 2 * DIM), jnp.float32),
)


def _make_gather_kernel():
    mesh = plsc.VectorSubcoreMesh(core_axis_name="c", subcore_axis_name="s")

    @functools.partial(
        pl.kernel,
        mesh=mesh,
        out_type=jax.ShapeDtypeStruct((SRC_ROWS, DIM), jnp.float32),
        compiler_params=pltpu.CompilerParams(use_tc_tiling_on_sc=False),
        scratch_types=[
            pltpu.VMEM((B_PER_W, 2, KTILES), jnp.int32),
            pltpu.VMEM((NBUF, 2 * KTILES, DIM), jnp.float32),
        ]
        + [pltpu.SemaphoreType.DMA] * NBUF
        + [pltpu.SemaphoreType.DMA] * NBUF,
    )
    def emb_kernel(idx_hbm, table_hbm, out_hbm, idx_v, rows_v, *sems):
        sem_g = sems[:NBUF]
        sem_o = sems[NBUF:]
        wid = lax.axis_index("s") * NC + lax.axis_index("c")
        b0 = wid * B_PER_W

        pltpu.sync_copy(idx_hbm.at[wid], idx_v)

        def gathers(j, u, start):
            for i in range(2):
                cp = pltpu.make_async_copy(
                    table_hbm.at[idx_v.at[j, i]],
                    rows_v.at[u, pl.ds(i * KTILES, KTILES)],
                    sem_g[u],
                )
                cp.start() if start else cp.wait()

        def outcp(j, u, start):
            cp = pltpu.make_async_copy(
                rows_v.at[u],
                out_hbm.at[pl.ds((b0 + j) * 2 * GP, 2 * KTILES)],
                sem_o[u],
            )
            cp.start() if start else cp.wait()

        for u in range(NBUF):
            gathers(u, u, True)

        def body(r, carry):
            j0 = r * NBUF
            for u in range(NBUF):
                gathers(j0 + u, u, False)
                outcp(j0 + u, u, True)
            for u in range(NBUF):
                outcp(j0 + u, u, False)
                gathers(j0 + NBUF + u, u, True)
            return carry

        lax.fori_loop(0, ROUNDS - 1, body, 0)

        j0 = (ROUNDS - 1) * NBUF
        for u in range(NBUF):
            gathers(j0 + u, u, False)
            outcp(j0 + u, u, True)
        for u in range(NBUF):
            outcp(j0 + u, u, False)

    return emb_kernel


_emb_gather = _make_gather_kernel()


def _relayout_body(src_ref, out_ref):
    for k in range(KTILES):
        out_ref[:, k * 128 : (k + 1) * 128] = src_ref[pl.Slice(k, 8, GP), :]


_relayout = pl.pallas_call(
    _relayout_body,
    grid=(B // 8,),
    in_specs=[pl.BlockSpec((8 * GP, 128), lambda g: (g, 0))],
    out_specs=pl.BlockSpec((8, L * DIM), lambda g: (g, 0)),
    out_shape=jax.ShapeDtypeStruct((B, L * DIM), jnp.float32),
)


def kernel(x_input, table):
    packed = _pack_table(table.T)
    table_lin = packed.reshape(VPAD, DIM)
    v = x_input.reshape(N).astype(jnp.int32)
    f = (v & ~(WBLK - 1)) | ((v & (HBLK - 1)) << 1) | ((v // HBLK) & 1)
    idx = f.reshape(NW, B_PER_W, 2, KTILES)
    src = _emb_gather(idx, table_lin)
    return _relayout(src.reshape(SRC_ROWS * DIM // 128, 128))


# speedup vs baseline: 1.3033x; 1.3033x over previous
import functools

import jax
import jax.numpy as jnp
from jax import lax
from jax.experimental import pallas as pl
from jax.experimental.pallas import tpu as pltpu
from jax.experimental.pallas import tpu_sc as plsc

VOCAB = 1000000
DIM = 64
B = 4096
L = 200
N = B * L

WBLK = 16384
HBLK = WBLK // 2
NBLK = (VOCAB + WBLK - 1) // WBLK
VPAD = NBLK * WBLK

_info = plsc.get_sparse_core_info()
NC, NS = _info.num_cores, _info.num_subcores
NW = NC * NS
N_PER_W = N // NW
CHUNK = 128
STEPS = N_PER_W // CHUNK
NBUF = 4
ROUNDS = STEPS // NBUF


def _transpose_body(tt_ref, out_ref):
    a = tt_ref[:, :HBLK]
    b = tt_ref[:, HBLK:]
    c = jnp.concatenate([a, b], axis=0)
    out_ref[...] = c.T


_pack_table = pl.pallas_call(
    _transpose_body,
    grid=(NBLK,),
    in_specs=[pl.BlockSpec((DIM, WBLK), lambda q: (0, q))],
    out_specs=pl.BlockSpec((HBLK, 2 * DIM), lambda q: (q, 0)),
    out_shape=jax.ShapeDtypeStruct((VPAD // 2, 2 * DIM), jnp.float32),
)


def _make_gather_kernel():
    mesh = plsc.VectorSubcoreMesh(core_axis_name="c", subcore_axis_name="s")

    @functools.partial(
        pl.kernel,
        mesh=mesh,
        out_type=jax.ShapeDtypeStruct((N, DIM), jnp.float32),
        compiler_params=pltpu.CompilerParams(use_tc_tiling_on_sc=False),
        scratch_types=[
            pltpu.VMEM((STEPS, CHUNK), jnp.int32),
            pltpu.VMEM((NBUF, CHUNK, DIM), jnp.float32),
        ]
        + [pltpu.SemaphoreType.DMA] * NBUF
        + [pltpu.SemaphoreType.DMA] * NBUF,
    )
    def emb_kernel(idx_hbm, table_hbm, out_hbm, idx_v, rows_v, *sems):
        sem_g = sems[:NBUF]
        sem_o = sems[NBUF:]
        wid = lax.axis_index("s") * NC + lax.axis_index("c")
        base = wid * N_PER_W

        pltpu.sync_copy(idx_hbm.at[wid], idx_v)

        def gather(j, u, start):
            cp = pltpu.make_async_copy(
                table_hbm.at[idx_v.at[j]], rows_v.at[u], sem_g[u]
            )
            cp.start() if start else cp.wait()

        def outcp(j, u, start):
            cp = pltpu.make_async_copy(
                rows_v.at[u], out_hbm.at[pl.ds(base + j * CHUNK, CHUNK)], sem_o[u]
            )
            cp.start() if start else cp.wait()

        for u in range(NBUF):
            gather(u, u, True)

        def body(r, carry):
            j0 = r * NBUF
            for u in range(NBUF):
                gather(j0 + u, u, False)
                outcp(j0 + u, u, True)
            for u in range(NBUF):
                outcp(j0 + u, u, False)
                gather(j0 + NBUF + u, u, True)
            return carry

        lax.fori_loop(0, ROUNDS - 1, body, 0)

        j0 = (ROUNDS - 1) * NBUF
        for u in range(NBUF):
            gather(j0 + u, u, False)
            outcp(j0 + u, u, True)
        for u in range(NBUF):
            outcp(j0 + u, u, False)

    return emb_kernel


_emb_gather = _make_gather_kernel()


def kernel(x_input, table):
    packed = _pack_table(table.T)
    table_lin = packed.reshape(VPAD, DIM)
    v = x_input.reshape(N).astype(jnp.int32)
    f = (v & ~(WBLK - 1)) | ((v & (HBLK - 1)) << 1) | ((v // HBLK) & 1)
    idx = f.reshape(NW, STEPS, CHUNK)
    out = _emb_gather(idx, table_lin)
    return out.reshape(B, L * DIM)


# speedup vs baseline: 1.3215x; 1.0139x over previous
import functools

import jax
import jax.numpy as jnp
from jax import lax
from jax.experimental import pallas as pl
from jax.experimental.pallas import tpu as pltpu
from jax.experimental.pallas import tpu_sc as plsc

VOCAB = 1000000
DIM = 64
B = 4096
L = 200
N = B * L

WBLK = 32768
HBLK = WBLK // 2
NBLK = (VOCAB + WBLK - 1) // WBLK
VPAD = NBLK * WBLK

_info = plsc.get_sparse_core_info()
NC, NS = _info.num_cores, _info.num_subcores
NW = NC * NS
N_PER_W = N // NW
CHUNK = 128
STEPS = N_PER_W // CHUNK
NBUF = 8
ROUNDS = STEPS // NBUF


def _transpose_body(tt_ref, out_ref):
    a = tt_ref[:, :HBLK]
    b = tt_ref[:, HBLK:]
    c = jnp.concatenate([a, b], axis=0)
    out_ref[...] = c.T


_pack_table = pl.pallas_call(
    _transpose_body,
    grid=(NBLK,),
    in_specs=[pl.BlockSpec((DIM, WBLK), lambda q: (0, q))],
    out_specs=pl.BlockSpec((HBLK, 2 * DIM), lambda q: (q, 0)),
    out_shape=jax.ShapeDtypeStruct((VPAD // 2, 2 * DIM), jnp.float32),
)


def _make_gather_kernel():
    mesh = plsc.VectorSubcoreMesh(core_axis_name="c", subcore_axis_name="s")

    @functools.partial(
        pl.kernel,
        mesh=mesh,
        out_type=jax.ShapeDtypeStruct((N, DIM), jnp.float32),
        compiler_params=pltpu.CompilerParams(use_tc_tiling_on_sc=False),
        scratch_types=[
            pltpu.VMEM((STEPS, CHUNK), jnp.int32),
            pltpu.VMEM((NBUF, CHUNK, DIM), jnp.float32),
        ]
        + [pltpu.SemaphoreType.DMA] * NBUF
        + [pltpu.SemaphoreType.DMA] * NBUF,
    )
    def emb_kernel(idx_hbm, table_hbm, out_hbm, idx_v, rows_v, *sems):
        sem_g = sems[:NBUF]
        sem_o = sems[NBUF:]
        wid = lax.axis_index("s") * NC + lax.axis_index("c")
        base = wid * N_PER_W

        pltpu.sync_copy(idx_hbm.at[wid], idx_v)

        def gather(j, u, start):
            cp = pltpu.make_async_copy(
                table_hbm.at[idx_v.at[j]], rows_v.at[u], sem_g[u]
            )
            cp.start() if start else cp.wait()

        def outcp(j, u, start):
            cp = pltpu.make_async_copy(
                rows_v.at[u], out_hbm.at[pl.ds(base + j * CHUNK, CHUNK)], sem_o[u]
            )
            cp.start() if start else cp.wait()

        for u in range(NBUF):
            gather(u, u, True)

        def body(r, carry):
            j0 = r * NBUF
            for u in range(NBUF):
                gather(j0 + u, u, False)
                outcp(j0 + u, u, True)
            for u in range(NBUF):
                outcp(j0 + u, u, False)
                gather(j0 + NBUF + u, u, True)
            return carry

        lax.fori_loop(0, ROUNDS - 1, body, 0)

        j0 = (ROUNDS - 1) * NBUF
        for u in range(NBUF):
            gather(j0 + u, u, False)
            outcp(j0 + u, u, True)
        for u in range(NBUF):
            outcp(j0 + u, u, False)

    return emb_kernel


_emb_gather = _make_gather_kernel()


def kernel(x_input, table):
    packed = _pack_table(table.T)
    table_lin = packed.reshape(VPAD, DIM)
    v = x_input.reshape(N).astype(jnp.int32)
    f = (v & ~(WBLK - 1)) | ((v & (HBLK - 1)) << 1) | ((v // HBLK) & 1)
    idx = f.reshape(NW, STEPS, CHUNK)
    out = _emb_gather(idx, table_lin)
    return out.reshape(B, L * DIM)
